# Initial kernel scaffold; baseline (speedup 1.0000x reference)
import functools

import jax
import jax.numpy as jnp
from jax import lax
from jax.experimental import pallas as pl
from jax.experimental.pallas import tpu as pltpu
from jax.experimental.pallas import tpu_sc as plsc

N = 10000
E = 320000
D = 128

NC = 2
NS = 16
CH = 128

NPAD = 10240
EP = ((E + NC * NS * CH - 1) // (NC * NS * CH)) * (NC * NS * CH)
AGG_CH = EP // (NC * NS * CH)
DEG_CH = EP // (NS * CH)
ROWS_PER = NPAD // NS

BR = 512

_mesh = plsc.VectorSubcoreMesh(core_axis_name="c", subcore_axis_name="s")


@functools.partial(
    pl.kernel,
    mesh=_mesh,
    out_type=jax.ShapeDtypeStruct((NC, NPAD, 16), jnp.float32),
    scratch_types=[
        pltpu.VMEM((DEG_CH, CH), jnp.int32),
        pltpu.VMEM((CH, 16), jnp.float32),
        pltpu.VMEM_SHARED((NPAD, 16), jnp.float32),
    ],
)
def _deg_kernel(idx_hbm, zeros_hbm, ones_hbm, out_hbm, idx_v, ones_v, cnt_sh):
    cid = lax.axis_index("c")
    sid = lax.axis_index("s")
    pltpu.sync_copy(zeros_hbm, cnt_sh.at[pl.ds(sid * ROWS_PER, ROWS_PER)])
    pltpu.sync_copy(idx_hbm.at[cid, sid], idx_v)
    pltpu.sync_copy(ones_hbm, ones_v)
    plsc.subcore_barrier()

    @pl.loop(0, DEG_CH)
    def _(j):
        pltpu.sync_copy(ones_v, cnt_sh.at[idx_v.at[j]], add=True)

    plsc.subcore_barrier()
    pltpu.sync_copy(
        cnt_sh.at[pl.ds(sid * ROWS_PER, ROWS_PER)],
        out_hbm.at[cid, pl.ds(sid * ROWS_PER, ROWS_PER)],
    )


@functools.partial(
    pl.kernel,
    mesh=_mesh,
    out_type=jax.ShapeDtypeStruct((NC, NPAD, D), jnp.float32),
    scratch_types=[
        pltpu.VMEM((AGG_CH, CH), jnp.int32),
        pltpu.VMEM((AGG_CH, CH), jnp.int32),
        pltpu.VMEM((CH, D), jnp.float32),
        pltpu.VMEM_SHARED((NPAD, D), jnp.float32),
        pltpu.SemaphoreType.DMA,
    ],
)
def _agg_kernel(hs_hbm, src_hbm, dst_hbm, zeros_hbm, out_hbm,
                src_v, dst_v, rows_v, agg_sh, sem):
    cid = lax.axis_index("c")
    sid = lax.axis_index("s")
    pltpu.sync_copy(zeros_hbm, agg_sh.at[pl.ds(sid * ROWS_PER, ROWS_PER)])
    pltpu.sync_copy(src_hbm.at[cid, sid], src_v)
    pltpu.sync_copy(dst_hbm.at[cid, sid], dst_v)
    plsc.subcore_barrier()

    @pl.loop(0, AGG_CH)
    def _(j):
        pltpu.async_copy(hs_hbm.at[src_v.at[j]], rows_v, sem).wait()
        pltpu.sync_copy(rows_v, agg_sh.at[dst_v.at[j]], add=True)

    plsc.subcore_barrier()
    pltpu.sync_copy(
        agg_sh.at[pl.ds(sid * ROWS_PER, ROWS_PER)],
        out_hbm.at[cid, pl.ds(sid * ROWS_PER, ROWS_PER)],
    )


def _prep_body(cs_ref, cd_ref, feat_ref, cb_ref, rb_ref, hs_ref):
    c = lax.rsqrt(jnp.maximum(cs_ref[:, 0:1], 1.0))
    r = lax.rsqrt(jnp.maximum(cd_ref[:, 0:1], 1.0))
    cb = jnp.broadcast_to(c, (BR, D))
    cb_ref[...] = cb
    rb_ref[...] = jnp.broadcast_to(r, (BR, D))
    hs_ref[...] = feat_ref[...] * cb


def _prep_tc(cs, cd, featp):
    return pl.pallas_call(
        _prep_body,
        grid=(NPAD // BR,),
        in_specs=[
            pl.BlockSpec((BR, 16), lambda i: (i, 0)),
            pl.BlockSpec((BR, 16), lambda i: (i, 0)),
            pl.BlockSpec((BR, D), lambda i: (i, 0)),
        ],
        out_specs=[
            pl.BlockSpec((BR, D), lambda i: (i, 0)),
            pl.BlockSpec((BR, D), lambda i: (i, 0)),
            pl.BlockSpec((BR, D), lambda i: (i, 0)),
        ],
        out_shape=[jax.ShapeDtypeStruct((NPAD, D), jnp.float32)] * 3,
    )(cs, cd, featp)


def _layer_body(relu, want_hs, aggs_ref, rb_ref, hp_ref, cb_ref, w_ref,
                p_ref, h_ref, *maybe_hs_ref):
    x = (aggs_ref[0] + aggs_ref[1]) * rb_ref[...]
    m = jnp.dot(x, w_ref[...], preferred_element_type=jnp.float32)
    gs = p_ref[0:1, :]
    b2 = p_ref[1:2, :]
    y = (m + hp_ref[...]) * gs + b2
    if relu:
        y = jnp.maximum(y, 0.0)
    h_ref[...] = y
    if want_hs:
        maybe_hs_ref[0][...] = y * cb_ref[...]


def _layer_tc(aggs, rb, hp, cb, w, p, relu, want_hs):
    n_out = 2 if want_hs else 1
    return pl.pallas_call(
        functools.partial(_layer_body, relu, want_hs),
        grid=(NPAD // BR,),
        in_specs=[
            pl.BlockSpec((NC, BR, D), lambda i: (0, i, 0)),
            pl.BlockSpec((BR, D), lambda i: (i, 0)),
            pl.BlockSpec((BR, D), lambda i: (i, 0)),
            pl.BlockSpec((BR, D), lambda i: (i, 0)),
            pl.BlockSpec((D, D), lambda i: (0, 0)),
            pl.BlockSpec((8, D), lambda i: (0, 0)),
        ],
        out_specs=[pl.BlockSpec((BR, D), lambda i: (i, 0))] * n_out,
        out_shape=[jax.ShapeDtypeStruct((NPAD, D), jnp.float32)] * n_out,
    )(aggs, rb, hp, cb, w, p)


def kernel(feat, edge_index, W1, b1, W2, b2, W3, b3, g1, be1, g2, be2):
    src = edge_index[0]
    dst = edge_index[1]
    pad = EP - E
    srcp = jnp.concatenate([src, jnp.full((pad,), N, jnp.int32)])
    dstp = jnp.concatenate([dst, jnp.full((pad,), N, jnp.int32)])
    deg_idx = jnp.stack([srcp.reshape(NS, DEG_CH, CH),
                         dstp.reshape(NS, DEG_CH, CH)])
    src4 = srcp.reshape(NC, NS, AGG_CH, CH)
    dst4 = dstp.reshape(NC, NS, AGG_CH, CH)

    zeros16 = jnp.zeros((ROWS_PER, 16), jnp.float32)
    ones16 = jnp.ones((CH, 16), jnp.float32)
    zerosD = jnp.zeros((ROWS_PER, D), jnp.float32)

    cnts = _deg_kernel(deg_idx, zeros16, ones16)
    featp = jnp.pad(feat, ((0, NPAD - N), (0, 0)))
    cb, rb, hs = _prep_tc(cnts[0], cnts[1], featp)

    inv = 1.0 / jnp.sqrt(jnp.float32(1.0 + 1e-5))
    gs1 = g1 * inv
    p1 = jnp.zeros((8, D), jnp.float32).at[0].set(gs1).at[1].set(b1 * gs1 + be1)
    gs2 = g2 * inv
    p2 = jnp.zeros((8, D), jnp.float32).at[0].set(gs2).at[1].set(b2 * gs2 + be2)
    p3 = jnp.zeros((8, D), jnp.float32).at[0].set(jnp.ones((D,))).at[1].set(b3)

    h = featp
    aggs = _agg_kernel(hs, src4, dst4, zerosD)
    h, hs = _layer_tc(aggs, rb, h, cb, W1, p1, relu=True, want_hs=True)
    aggs = _agg_kernel(hs, src4, dst4, zerosD)
    h, hs = _layer_tc(aggs, rb, h, cb, W2, p2, relu=True, want_hs=True)
    aggs = _agg_kernel(hs, src4, dst4, zerosD)
    (h,) = _layer_tc(aggs, rb, h, cb, W3, p3, relu=False, want_hs=False)
    return h[:N]


# speedup vs baseline: 3.7921x; 3.7921x over previous
import functools

import jax
import jax.numpy as jnp
from jax import lax
from jax.experimental import pallas as pl
from jax.experimental.pallas import tpu as pltpu
from jax.experimental.pallas import tpu_sc as plsc

N = 10000
E = 320000
D = 128

NC = 2
NS = 16
CH = 128

NPAD = 10240
EP = ((E + NC * NS * CH - 1) // (NC * NS * CH)) * (NC * NS * CH)
AGG_CH = EP // (NC * NS * CH)
DEG_CH = EP // (NS * CH)
ROWS_PER = NPAD // NS

BR = 512

_mesh = plsc.VectorSubcoreMesh(core_axis_name="c", subcore_axis_name="s")


@functools.partial(
    pl.kernel,
    mesh=_mesh,
    out_type=jax.ShapeDtypeStruct((NC, NPAD, D), jnp.float32),
    scratch_types=[
        pltpu.VMEM((DEG_CH, CH), jnp.int32),
        pltpu.VMEM((CH, D), jnp.float32),
        pltpu.VMEM_SHARED((NPAD, D), jnp.float32),
    ],
)
def _deg_kernel(idx_hbm, zeros_hbm, ones_hbm, out_hbm, idx_v, ones_v, cnt_sh):
    cid = lax.axis_index("c")
    sid = lax.axis_index("s")
    pltpu.sync_copy(zeros_hbm, cnt_sh.at[pl.ds(sid * ROWS_PER, ROWS_PER)])
    pltpu.sync_copy(idx_hbm.at[cid, sid], idx_v)
    pltpu.sync_copy(ones_hbm, ones_v)
    plsc.subcore_barrier()

    @pl.loop(0, DEG_CH)
    def _(j):
        pltpu.sync_copy(ones_v, cnt_sh.at[idx_v.at[j]], add=True)

    plsc.subcore_barrier()
    pltpu.sync_copy(
        cnt_sh.at[pl.ds(sid * ROWS_PER, ROWS_PER)],
        out_hbm.at[cid, pl.ds(sid * ROWS_PER, ROWS_PER)],
    )


@functools.partial(
    pl.kernel,
    mesh=_mesh,
    out_type=jax.ShapeDtypeStruct((NC, NPAD, D), jnp.float32),
    scratch_types=[
        pltpu.VMEM((AGG_CH, CH), jnp.int32),
        pltpu.VMEM((AGG_CH, CH), jnp.int32),
        pltpu.VMEM((CH, D), jnp.float32),
        pltpu.VMEM_SHARED((NPAD, D), jnp.float32),
        pltpu.SemaphoreType.DMA,
    ],
)
def _agg_kernel(hs_hbm, src_hbm, dst_hbm, zeros_hbm, out_hbm,
                src_v, dst_v, rows_v, agg_sh, sem):
    cid = lax.axis_index("c")
    sid = lax.axis_index("s")
    pltpu.sync_copy(zeros_hbm, agg_sh.at[pl.ds(sid * ROWS_PER, ROWS_PER)])
    pltpu.sync_copy(src_hbm.at[cid, sid], src_v)
    pltpu.sync_copy(dst_hbm.at[cid, sid], dst_v)
    plsc.subcore_barrier()

    @pl.loop(0, AGG_CH)
    def _(j):
        pltpu.async_copy(hs_hbm.at[src_v.at[j]], rows_v, sem).wait()
        pltpu.sync_copy(rows_v, agg_sh.at[dst_v.at[j]], add=True)

    plsc.subcore_barrier()
    pltpu.sync_copy(
        agg_sh.at[pl.ds(sid * ROWS_PER, ROWS_PER)],
        out_hbm.at[cid, pl.ds(sid * ROWS_PER, ROWS_PER)],
    )


def _prep_body(cs_ref, cd_ref, feat_ref, cb_ref, rb_ref, hs_ref):
    cb = lax.rsqrt(jnp.maximum(cs_ref[...], 1.0))
    cb_ref[...] = cb
    rb_ref[...] = lax.rsqrt(jnp.maximum(cd_ref[...], 1.0))
    hs_ref[...] = feat_ref[...] * cb


def _prep_tc(cs, cd, featp):
    return pl.pallas_call(
        _prep_body,
        grid=(NPAD // BR,),
        in_specs=[
            pl.BlockSpec((BR, D), lambda i: (i, 0)),
            pl.BlockSpec((BR, D), lambda i: (i, 0)),
            pl.BlockSpec((BR, D), lambda i: (i, 0)),
        ],
        out_specs=[
            pl.BlockSpec((BR, D), lambda i: (i, 0)),
            pl.BlockSpec((BR, D), lambda i: (i, 0)),
            pl.BlockSpec((BR, D), lambda i: (i, 0)),
        ],
        out_shape=[jax.ShapeDtypeStruct((NPAD, D), jnp.float32)] * 3,
    )(cs, cd, featp)


def _layer_body(relu, want_hs, aggs_ref, rb_ref, hp_ref, cb_ref, w_ref,
                p_ref, h_ref, *maybe_hs_ref):
    x = (aggs_ref[0] + aggs_ref[1]) * rb_ref[...]
    m = jnp.dot(x, w_ref[...], preferred_element_type=jnp.float32)
    gs = p_ref[0:1, :]
    b2 = p_ref[1:2, :]
    y = (m + hp_ref[...]) * gs + b2
    if relu:
        y = jnp.maximum(y, 0.0)
    h_ref[...] = y
    if want_hs:
        maybe_hs_ref[0][...] = y * cb_ref[...]


def _layer_tc(aggs, rb, hp, cb, w, p, relu, want_hs):
    n_out = 2 if want_hs else 1
    return pl.pallas_call(
        functools.partial(_layer_body, relu, want_hs),
        grid=(NPAD // BR,),
        in_specs=[
            pl.BlockSpec((NC, BR, D), lambda i: (0, i, 0)),
            pl.BlockSpec((BR, D), lambda i: (i, 0)),
            pl.BlockSpec((BR, D), lambda i: (i, 0)),
            pl.BlockSpec((BR, D), lambda i: (i, 0)),
            pl.BlockSpec((D, D), lambda i: (0, 0)),
            pl.BlockSpec((8, D), lambda i: (0, 0)),
        ],
        out_specs=[pl.BlockSpec((BR, D), lambda i: (i, 0))] * n_out,
        out_shape=[jax.ShapeDtypeStruct((NPAD, D), jnp.float32)] * n_out,
    )(aggs, rb, hp, cb, w, p)


def kernel(feat, edge_index, W1, b1, W2, b2, W3, b3, g1, be1, g2, be2):
    src = edge_index[0]
    dst = edge_index[1]
    pad = EP - E
    srcp = jnp.concatenate([src, jnp.full((pad,), N, jnp.int32)])
    dstp = jnp.concatenate([dst, jnp.full((pad,), N, jnp.int32)])
    deg_idx = jnp.stack([srcp.reshape(NS, DEG_CH, CH),
                         dstp.reshape(NS, DEG_CH, CH)])
    src4 = srcp.reshape(NC, NS, AGG_CH, CH)
    dst4 = dstp.reshape(NC, NS, AGG_CH, CH)

    zerosD = jnp.zeros((ROWS_PER, D), jnp.float32)
    onesD = jnp.ones((CH, D), jnp.float32)

    cnts = _deg_kernel(deg_idx, zerosD, onesD)
    featp = jnp.pad(feat, ((0, NPAD - N), (0, 0)))
    cb, rb, hs = _prep_tc(cnts[0], cnts[1], featp)

    inv = 1.0 / jnp.sqrt(jnp.float32(1.0 + 1e-5))
    gs1 = g1 * inv
    p1 = jnp.zeros((8, D), jnp.float32).at[0].set(gs1).at[1].set(b1 * gs1 + be1)
    gs2 = g2 * inv
    p2 = jnp.zeros((8, D), jnp.float32).at[0].set(gs2).at[1].set(b2 * gs2 + be2)
    p3 = jnp.zeros((8, D), jnp.float32).at[0].set(jnp.ones((D,))).at[1].set(b3)

    h = featp
    aggs = _agg_kernel(hs, src4, dst4, zerosD)
    h, hs = _layer_tc(aggs, rb, h, cb, W1, p1, relu=True, want_hs=True)
    aggs = _agg_kernel(hs, src4, dst4, zerosD)
    h, hs = _layer_tc(aggs, rb, h, cb, W2, p2, relu=True, want_hs=True)
    aggs = _agg_kernel(hs, src4, dst4, zerosD)
    (h,) = _layer_tc(aggs, rb, h, cb, W3, p3, relu=False, want_hs=False)
    return h[:N]
